# baseline (speedup 1.0000x reference)
import functools

import jax
import jax.numpy as jnp
from jax import lax
from jax.experimental import pallas as pl
from jax.experimental.pallas import tpu as pltpu
from jax.experimental.pallas import tpu_sc as plsc

N = 10000
E = 160000

NC = 2
NS = 16
LANES = 16

CHUNK = 64
CPW = 2 * -(-E // (NC * NS * CHUNK * 2))
EP = NC * NS * CPW * CHUNK
NPAD = 10240
ROWS_OUT = NPAD // NS
ZSLABS = NPAD // (NS * CHUNK)


def _spmm_sc(d):
    mesh = plsc.VectorSubcoreMesh(core_axis_name="c", subcore_axis_name="s")

    @functools.partial(
        pl.kernel,
        out_type=jax.ShapeDtypeStruct((NC, NPAD, d), jnp.float32),
        mesh=mesh,
        scratch_types=[
            pltpu.VMEM((CPW, CHUNK), jnp.int32),
            pltpu.VMEM((CPW, CHUNK), jnp.int32),
            pltpu.VMEM((2, CHUNK, d), jnp.float32),
            pltpu.VMEM_SHARED((NPAD, d), jnp.float32),
            pltpu.SemaphoreType.DMA,
            pltpu.SemaphoreType.DMA,
        ],
    )
    def spmm(hw_hbm, dst_hbm, src_hbm, out_hbm, dst_v, src_v, rows_v, acc_sh,
             gsem0, gsem1):
        cid = lax.axis_index("c")
        sid = lax.axis_index("s")
        gsems = (gsem0, gsem1)

        pltpu.sync_copy(dst_hbm.at[cid, sid], dst_v)
        pltpu.sync_copy(src_hbm.at[cid, sid], src_v)
        pltpu.async_copy(hw_hbm.at[dst_v.at[0]], rows_v.at[0], gsem0)

        zeros16 = jnp.zeros((LANES,), jnp.float32)

        def zbody(i, carry):
            for jj in range(d // LANES):
                rows_v[1, i, pl.ds(jj * LANES, LANES)] = zeros16
            return carry

        lax.fori_loop(0, CHUNK, zbody, 0)
        for k in range(ZSLABS):
            pltpu.sync_copy(
                rows_v.at[1], acc_sh.at[pl.ds((sid * ZSLABS + k) * CHUNK, CHUNK)]
            )
        plsc.subcore_barrier()

        def body(t, carry):
            for b in range(2):
                j = 2 * t + b
                nb = 1 - b

                @pl.when(j + 1 < CPW)
                def _():
                    pltpu.async_copy(
                        hw_hbm.at[dst_v.at[j + 1]], rows_v.at[nb], gsems[nb]
                    )

                pltpu.make_async_copy(
                    hw_hbm.at[dst_v.at[j]], rows_v.at[b], gsems[b]
                ).wait()
                pltpu.sync_copy(rows_v.at[b], acc_sh.at[src_v.at[j]], add=True)
            return carry

        lax.fori_loop(0, CPW // 2, body, 0)
        plsc.subcore_barrier()

        pltpu.sync_copy(
            acc_sh.at[pl.ds(sid * ROWS_OUT, ROWS_OUT)],
            out_hbm.at[cid].at[pl.ds(sid * ROWS_OUT, ROWS_OUT)],
        )

    return spmm


_BLK = 1000


def _mm_body(h_ref, w_ref, o_ref):
    o_ref[...] = jnp.dot(
        h_ref[...], w_ref[...],
        preferred_element_type=jnp.float32,
    )


def _fused_body(z_ref, w_ref, o_ref):
    z = z_ref[0] + z_ref[1]
    h = jnp.where(z > 0, z, jnp.exp(z) - 1.0)
    o_ref[...] = jnp.dot(
        h, w_ref[...],
        preferred_element_type=jnp.float32,
    )


def _final_body(z_ref, o_ref):
    z = z_ref[0, :, :64] + z_ref[1, :, :64]
    h = jnp.where(z > 0, z, jnp.exp(z) - 1.0)
    m = jnp.max(h, axis=1, keepdims=True)
    e = jnp.exp(h - m)
    lse = jnp.log(jnp.sum(e, axis=1, keepdims=True))
    o_ref[...] = h - m - lse


def _mm(h, w):
    n, din = h.shape
    dout = w.shape[1]
    return pl.pallas_call(
        _mm_body,
        out_shape=jax.ShapeDtypeStruct((n, dout), jnp.float32),
        grid=(n // _BLK,),
        in_specs=[
            pl.BlockSpec((_BLK, din), lambda i: (i, 0)),
            pl.BlockSpec((din, dout), lambda i: (0, 0)),
        ],
        out_specs=pl.BlockSpec((_BLK, dout), lambda i: (i, 0)),
    )(h, w)


def _fused_mm(zp, w):
    _, _, din = zp.shape
    dout = w.shape[1]
    return pl.pallas_call(
        _fused_body,
        out_shape=jax.ShapeDtypeStruct((N, dout), jnp.float32),
        grid=(N // _BLK,),
        in_specs=[
            pl.BlockSpec((NC, _BLK, din), lambda i: (0, i, 0)),
            pl.BlockSpec((din, dout), lambda i: (0, 0)),
        ],
        out_specs=pl.BlockSpec((_BLK, dout), lambda i: (i, 0)),
    )(zp, w)


def _final(zp):
    _, _, d = zp.shape
    return pl.pallas_call(
        _final_body,
        out_shape=jax.ShapeDtypeStruct((N, 64), jnp.float32),
        grid=(N // _BLK,),
        in_specs=[pl.BlockSpec((NC, _BLK, d), lambda i: (0, i, 0))],
        out_specs=pl.BlockSpec((_BLK, 64), lambda i: (i, 0)),
    )(zp)


def kernel(features, edge_index, W0, W1, W2):
    src = edge_index[0].astype(jnp.int32)
    dst = edge_index[1].astype(jnp.int32)
    half = E // NC
    ppc = (EP - E) // NC
    padsrc = N + (jnp.arange(ppc, dtype=jnp.int32) % (NPAD - N))
    paddst = jnp.arange(ppc, dtype=jnp.int32) % N
    srcp = jnp.concatenate([src[:half], padsrc, src[half:], padsrc])
    dstp = jnp.concatenate([dst[:half], paddst, dst[half:], paddst])
    srcp = srcp.reshape(NC, NS, CPW, CHUNK)
    dstp = dstp.reshape(NC, NS, CPW, CHUNK)

    spmm128 = _spmm_sc(128)
    W2p = jnp.pad(W2, ((0, 0), (0, 128 - W2.shape[1])))

    hw = _mm(features, W0)
    zp = spmm128(hw, dstp, srcp)
    hw = _fused_mm(zp, W1)
    zp = spmm128(hw, dstp, srcp)
    hw = _fused_mm(zp, W2p)
    zp = spmm128(hw, dstp, srcp)
    return _final(zp)


# speedup vs baseline: 1.2147x; 1.2147x over previous
import functools

import jax
import jax.numpy as jnp
from jax import lax
from jax.experimental import pallas as pl
from jax.experimental.pallas import tpu as pltpu
from jax.experimental.pallas import tpu_sc as plsc

N = 10000
E = 160000

NC = 2
NS = 16
LANES = 16

CHUNK = 128
CPW = 2 * -(-E // (NC * NS * CHUNK * 2))
EP = NC * NS * CPW * CHUNK
NPAD = 10240
ROWS_OUT = NPAD // NS
ZSLABS = NPAD // (NS * CHUNK)


def _spmm_sc(d):
    mesh = plsc.VectorSubcoreMesh(core_axis_name="c", subcore_axis_name="s")

    @functools.partial(
        pl.kernel,
        out_type=jax.ShapeDtypeStruct((NC, NPAD, d), jnp.float32),
        mesh=mesh,
        scratch_types=[
            pltpu.VMEM((CPW, CHUNK), jnp.int32),
            pltpu.VMEM((CPW, CHUNK), jnp.int32),
            pltpu.VMEM((2, CHUNK, d), jnp.float32),
            pltpu.VMEM_SHARED((NPAD, d), jnp.float32),
            pltpu.SemaphoreType.DMA,
            pltpu.SemaphoreType.DMA,
        ],
    )
    def spmm(hw_hbm, dst_hbm, src_hbm, out_hbm, dst_v, src_v, rows_v, acc_sh,
             gsem0, gsem1):
        cid = lax.axis_index("c")
        sid = lax.axis_index("s")
        gsems = (gsem0, gsem1)

        pltpu.sync_copy(dst_hbm.at[cid, sid], dst_v)
        pltpu.sync_copy(src_hbm.at[cid, sid], src_v)
        pltpu.async_copy(hw_hbm.at[dst_v.at[0]], rows_v.at[0], gsem0)

        zeros16 = jnp.zeros((LANES,), jnp.float32)

        def zbody(i, carry):
            for jj in range(d // LANES):
                rows_v[1, i, pl.ds(jj * LANES, LANES)] = zeros16
            return carry

        lax.fori_loop(0, CHUNK, zbody, 0)
        for k in range(ZSLABS):
            pltpu.sync_copy(
                rows_v.at[1], acc_sh.at[pl.ds((sid * ZSLABS + k) * CHUNK, CHUNK)]
            )
        plsc.subcore_barrier()

        def body(t, carry):
            for b in range(2):
                j = 2 * t + b
                nb = 1 - b

                @pl.when(j + 1 < CPW)
                def _():
                    pltpu.async_copy(
                        hw_hbm.at[dst_v.at[j + 1]], rows_v.at[nb], gsems[nb]
                    )

                pltpu.make_async_copy(
                    hw_hbm.at[dst_v.at[j]], rows_v.at[b], gsems[b]
                ).wait()
                pltpu.sync_copy(rows_v.at[b], acc_sh.at[src_v.at[j]], add=True)
            return carry

        lax.fori_loop(0, CPW // 2, body, 0)
        plsc.subcore_barrier()

        pltpu.sync_copy(
            acc_sh.at[pl.ds(sid * ROWS_OUT, ROWS_OUT)],
            out_hbm.at[cid].at[pl.ds(sid * ROWS_OUT, ROWS_OUT)],
        )

    return spmm


_BLK = 2000


def _mm_body(h_ref, w_ref, o_ref):
    o_ref[...] = jnp.dot(
        h_ref[...], w_ref[...],
        preferred_element_type=jnp.float32,
    )


def _fused_body(z_ref, w_ref, o_ref):
    z = z_ref[0] + z_ref[1]
    h = jnp.where(z > 0, z, jnp.exp(z) - 1.0)
    o_ref[...] = jnp.dot(
        h, w_ref[...],
        preferred_element_type=jnp.float32,
    )


def _final_body(z_ref, o_ref):
    z = z_ref[0, :, :64] + z_ref[1, :, :64]
    h = jnp.where(z > 0, z, jnp.exp(z) - 1.0)
    m = jnp.max(h, axis=1, keepdims=True)
    e = jnp.exp(h - m)
    lse = jnp.log(jnp.sum(e, axis=1, keepdims=True))
    o_ref[...] = h - m - lse


def _mm(h, w):
    n, din = h.shape
    dout = w.shape[1]
    return pl.pallas_call(
        _mm_body,
        out_shape=jax.ShapeDtypeStruct((n, dout), jnp.float32),
        grid=(n // _BLK,),
        in_specs=[
            pl.BlockSpec((_BLK, din), lambda i: (i, 0)),
            pl.BlockSpec((din, dout), lambda i: (0, 0)),
        ],
        out_specs=pl.BlockSpec((_BLK, dout), lambda i: (i, 0)),
    )(h, w)


def _fused_mm(zp, w):
    _, _, din = zp.shape
    dout = w.shape[1]
    return pl.pallas_call(
        _fused_body,
        out_shape=jax.ShapeDtypeStruct((N, dout), jnp.float32),
        grid=(N // _BLK,),
        in_specs=[
            pl.BlockSpec((NC, _BLK, din), lambda i: (0, i, 0)),
            pl.BlockSpec((din, dout), lambda i: (0, 0)),
        ],
        out_specs=pl.BlockSpec((_BLK, dout), lambda i: (i, 0)),
    )(zp, w)


def _final(zp):
    _, _, d = zp.shape
    return pl.pallas_call(
        _final_body,
        out_shape=jax.ShapeDtypeStruct((N, 64), jnp.float32),
        grid=(N // _BLK,),
        in_specs=[pl.BlockSpec((NC, _BLK, d), lambda i: (0, i, 0))],
        out_specs=pl.BlockSpec((_BLK, 64), lambda i: (i, 0)),
    )(zp)


def kernel(features, edge_index, W0, W1, W2):
    src = edge_index[0].astype(jnp.int32)
    dst = edge_index[1].astype(jnp.int32)
    pad = EP - E
    padsrc = N + (jnp.arange(pad, dtype=jnp.int32) % (NPAD - N))
    paddst = jnp.arange(pad, dtype=jnp.int32) % N
    srcp = jnp.concatenate([src, padsrc])
    dstp = jnp.concatenate([dst, paddst])
    srcp = srcp.reshape(NC, NS, CPW, CHUNK)
    dstp = dstp.reshape(NC, NS, CPW, CHUNK)

    spmm128 = _spmm_sc(128)
    W2p = jnp.pad(W2, ((0, 0), (0, 128 - W2.shape[1])))

    hw = _mm(features, W0)
    zp = spmm128(hw, dstp, srcp)
    hw = _fused_mm(zp, W1)
    zp = spmm128(hw, dstp, srcp)
    hw = _fused_mm(zp, W2p)
    zp = spmm128(hw, dstp, srcp)
    return _final(zp)
